# baseline (speedup 1.0000x reference)
import functools

import jax
import jax.numpy as jnp
from jax import lax
from jax.experimental import pallas as pl
from jax.experimental.pallas import tpu as pltpu
from jax.experimental.pallas import tpu_sc as plsc

N_R, D = 100000, 128
NC, NS = 2, 16
NW = NC * NS
CH = 128
NCH = 25
B_PER_W = CH * NCH
N_PAD = NW * B_PER_W


@functools.lru_cache(maxsize=1)
def _make_gather():
    mesh = plsc.VectorSubcoreMesh(
        core_axis_name="c", subcore_axis_name="s", num_cores=NC, num_subcores=NS)

    @functools.partial(
        pl.kernel,
        out_type=(jax.ShapeDtypeStruct((N_PAD, D), jnp.float32),
                  jax.ShapeDtypeStruct((N_PAD, D), jnp.float32)),
        mesh=mesh,
        scratch_types=(
            [pltpu.VMEM((NCH, CH), jnp.int32)] * 2
            + [pltpu.VMEM((CH, D), jnp.float32)] * 6
            + [pltpu.SemaphoreType.DMA] * 12
        ),
    )
    def gather_k(tab_u, tab_i, idx_u, idx_i, out_u, out_i,
                 iv_u, iv_i, bu0, bu1, bu2, bi0, bi1, bi2,
                 gu0, gu1, gu2, gi0, gi1, gi2,
                 su0, su1, su2, si0, si1, si2):
        wid = lax.axis_index("c") * NS + lax.axis_index("s")
        base = wid * B_PER_W
        pltpu.sync_copy(idx_u.at[wid], iv_u)
        pltpu.sync_copy(idx_i.at[wid], iv_i)
        streams = (
            (tab_u, iv_u, out_u, (bu0, bu1, bu2), (gu0, gu1, gu2),
             (su0, su1, su2), [None] * 3, [None] * 3),
            (tab_i, iv_i, out_i, (bi0, bi1, bi2), (gi0, gi1, gi2),
             (si0, si1, si2), [None] * 3, [None] * 3),
        )
        for tab, iv, out, bufs, gs, ss, gcp, scp in streams:
            for j in range(NCH):
                b = j % 3
                if scp[b] is not None:
                    scp[b].wait()
                gcp[b] = pltpu.async_copy(tab.at[iv.at[j]], bufs[b], gs[b])
                if j >= 2:
                    pb = (j - 2) % 3
                    gcp[pb].wait()
                    scp[pb] = pltpu.async_copy(
                        bufs[pb], out.at[pl.ds(base + (j - 2) * CH, CH)], ss[pb])
            for j in (NCH - 2, NCH - 1):
                pb = j % 3
                gcp[pb].wait()
                scp[pb] = pltpu.async_copy(
                    bufs[pb], out.at[pl.ds(base + j * CH, CH)], ss[pb])
            for s in scp:
                if s is not None:
                    s.wait()

    return gather_k


BR = 1000


def _mm_body(r_ref, u_ref, i_ref, w_ref, o_ref):
    acc = jnp.dot(r_ref[...], w_ref[0:D, :], preferred_element_type=jnp.float32)
    acc += jnp.dot(u_ref[...], w_ref[D:2 * D, :], preferred_element_type=jnp.float32)
    acc += jnp.dot(i_ref[...], w_ref[2 * D:3 * D, :], preferred_element_type=jnp.float32)
    o_ref[...] = jnp.maximum(acc, 0.0)


def _matmul_relu(review, ru, ri, w):
    return pl.pallas_call(
        _mm_body,
        grid=(N_R // BR,),
        in_specs=[
            pl.BlockSpec((BR, D), lambda i: (i, 0)),
            pl.BlockSpec((BR, D), lambda i: (i, 0)),
            pl.BlockSpec((BR, D), lambda i: (i, 0)),
            pl.BlockSpec((3 * D, D), lambda i: (0, 0)),
        ],
        out_specs=pl.BlockSpec((BR, D), lambda i: (i, 0)),
        out_shape=jax.ShapeDtypeStruct((N_R, D), jnp.float32),
    )(review, ru, ri, w)


def kernel(review_vecs, user_vecs, item_vecs, review_item_adj, review_user_adj, con_agg_weights):
    perm_i = jax.random.permutation(jax.random.key(1), D)
    perm_u = jax.random.permutation(jax.random.key(2), D)
    wr = con_agg_weights[:D]
    wu = con_agg_weights[D:2 * D][jnp.argsort(perm_u)]
    wi = con_agg_weights[2 * D:][jnp.argsort(perm_i)]
    w = jnp.concatenate([wr, wu, wi], axis=0)

    def pad_idx(a):
        return jnp.zeros((N_PAD,), jnp.int32).at[:N_R].set(a).reshape(NW, NCH, CH)

    gather = _make_gather()
    ru, ri = gather(user_vecs, item_vecs,
                    pad_idx(review_user_adj), pad_idx(review_item_adj))
    return _matmul_relu(review_vecs, ru, ri, w)


# speedup vs baseline: 1.3231x; 1.3231x over previous
import functools

import jax
import jax.numpy as jnp
from jax import lax
from jax.experimental import pallas as pl
from jax.experimental.pallas import tpu as pltpu
from jax.experimental.pallas import tpu_sc as plsc

N_R, D = 100000, 128
NC, NS = 2, 16
NW = NC * NS
CH = 128
NCH0 = 35
NCH1 = 15
NCHT = NS * (NCH0 + NCH1)
N_PAD = NCHT * CH


@functools.lru_cache(maxsize=1)
def _make_gather():
    mesh = plsc.VectorSubcoreMesh(
        core_axis_name="c", subcore_axis_name="s", num_cores=NC, num_subcores=NS)

    @functools.partial(
        pl.kernel,
        out_type=(jax.ShapeDtypeStruct((N_PAD, D), jnp.float32),
                  jax.ShapeDtypeStruct((N_PAD, D), jnp.float32)),
        mesh=mesh,
        scratch_types=(
            [pltpu.VMEM((NCH0 * CH,), jnp.int32)] * 2
            + [pltpu.VMEM((CH, D), jnp.float32)] * 6
            + [pltpu.SemaphoreType.DMA] * 12
        ),
    )
    def gather_k(tab_u, tab_i, idx_u, idx_i, out_u, out_i,
                 iv_u, iv_i, bu0, bu1, bu2, bi0, bi1, bi2,
                 gu0, gu1, gu2, gi0, gi1, gi2,
                 su0, su1, su2, si0, si1, si2):
        c = lax.axis_index("c")
        s = lax.axis_index("s")

        def run(nch, chunk0):
            base = chunk0 * CH
            pltpu.sync_copy(idx_u.at[pl.ds(base, nch * CH)],
                            iv_u.at[pl.ds(0, nch * CH)])
            pltpu.sync_copy(idx_i.at[pl.ds(base, nch * CH)],
                            iv_i.at[pl.ds(0, nch * CH)])
            streams = (
                (tab_u, iv_u, out_u, (bu0, bu1, bu2), (gu0, gu1, gu2),
                 (su0, su1, su2), [None] * 3, [None] * 3),
                (tab_i, iv_i, out_i, (bi0, bi1, bi2), (gi0, gi1, gi2),
                 (si0, si1, si2), [None] * 3, [None] * 3),
            )
            for j in range(nch):
                b = j % 3
                for tab, iv, out, bufs, gs, ss, gcp, scp in streams:
                    if scp[b] is not None:
                        scp[b].wait()
                    gcp[b] = pltpu.async_copy(
                        tab.at[iv.at[pl.ds(j * CH, CH)]], bufs[b], gs[b])
                if j >= 2:
                    pb = (j - 2) % 3
                    for tab, iv, out, bufs, gs, ss, gcp, scp in streams:
                        gcp[pb].wait()
                        scp[pb] = pltpu.async_copy(
                            bufs[pb], out.at[pl.ds(base + (j - 2) * CH, CH)], ss[pb])
            for j in (nch - 2, nch - 1):
                pb = j % 3
                for tab, iv, out, bufs, gs, ss, gcp, scp in streams:
                    gcp[pb].wait()
                    scp[pb] = pltpu.async_copy(
                        bufs[pb], out.at[pl.ds(base + j * CH, CH)], ss[pb])
            for tab, iv, out, bufs, gs, ss, gcp, scp in streams:
                for sc in scp:
                    if sc is not None:
                        sc.wait()

        @pl.when(c == 0)
        def _():
            run(NCH0, s * NCH0)

        @pl.when(c == 1)
        def _():
            run(NCH1, NS * NCH0 + s * NCH1)

    return gather_k


BR = 1000


def _mm_body(r_ref, u_ref, i_ref, w_ref, o_ref):
    acc = jnp.dot(r_ref[...], w_ref[0:D, :], preferred_element_type=jnp.float32)
    acc += jnp.dot(u_ref[...], w_ref[D:2 * D, :], preferred_element_type=jnp.float32)
    acc += jnp.dot(i_ref[...], w_ref[2 * D:3 * D, :], preferred_element_type=jnp.float32)
    o_ref[...] = jnp.maximum(acc, 0.0)


def _matmul_relu(review, ru, ri, w):
    return pl.pallas_call(
        _mm_body,
        grid=(N_R // BR,),
        in_specs=[
            pl.BlockSpec((BR, D), lambda i: (i, 0)),
            pl.BlockSpec((BR, D), lambda i: (i, 0)),
            pl.BlockSpec((BR, D), lambda i: (i, 0)),
            pl.BlockSpec((3 * D, D), lambda i: (0, 0)),
        ],
        out_specs=pl.BlockSpec((BR, D), lambda i: (i, 0)),
        out_shape=jax.ShapeDtypeStruct((N_R, D), jnp.float32),
    )(review, ru, ri, w)


def kernel(review_vecs, user_vecs, item_vecs, review_item_adj, review_user_adj, con_agg_weights):
    perm_i = jax.random.permutation(jax.random.key(1), D)
    perm_u = jax.random.permutation(jax.random.key(2), D)
    wr = con_agg_weights[:D]
    wu = con_agg_weights[D:2 * D][jnp.argsort(perm_u)]
    wi = con_agg_weights[2 * D:][jnp.argsort(perm_i)]
    w = jnp.concatenate([wr, wu, wi], axis=0)

    def pad_idx(a):
        return jnp.zeros((N_PAD,), jnp.int32).at[:N_R].set(a)

    gather = _make_gather()
    ru, ri = gather(user_vecs, item_vecs,
                    pad_idx(review_user_adj), pad_idx(review_item_adj))
    return _matmul_relu(review_vecs, ru, ri, w)
